# baseline (speedup 1.0000x reference)
import functools

import jax
import jax.numpy as jnp
from jax import lax
from jax.experimental import pallas as pl
from jax.experimental.pallas import tpu as pltpu
from jax.experimental.pallas import tpu_sc as plsc

NUM_ET = 4
N_NODES = 100000
N_EDGES = 150000
IN_DIM = 128

NW = 32
CB = 80
NBLK_ET = N_EDGES // CB
NB = NUM_ET * NBLK_ET
NG = CB // 16


def _sc_kernel_body(z_all, idx_hbm, w_hbm, sig_hbm, out_hbm,
                    idx0, idx1, rs0, rd0, rs1, rd1, ov0, ov1, w_v, sig_v,
                    si0, si1, sr0, sr1, so0, so1):
    cid = lax.axis_index("c")
    sid = lax.axis_index("s")
    wid = sid * 2 + cid

    pltpu.sync_copy(w_hbm, w_v)
    pltpu.sync_copy(sig_hbm, sig_v)
    sig = sig_v[...]
    lane = lax.iota(jnp.int32, 16)
    rids = [lane + g * 16 for g in range(NG)]

    n = (NB - wid + NW - 1) // NW

    def idx_copy(i, idxb, sem):
        b = wid + i * NW
        return pltpu.make_async_copy(
            idx_hbm.at[pl.ds(b * 2, 2)], idxb, sem)

    def gather_copies(idxb, rs, rd, sem):
        return (pltpu.make_async_copy(z_all.at[idxb.at[0]], rs, sem),
                pltpu.make_async_copy(z_all.at[idxb.at[1]], rd, sem))

    def out_copy(i, ov, sem):
        b = wid + i * NW
        return pltpu.make_async_copy(ov, out_hbm.at[pl.ds(b * CB, CB)], sem)

    def gather_start(idxb, rs, rd, sem):
        c1, c2 = gather_copies(idxb, rs, rd, sem)
        c1.start()
        c2.start()

    def gather_wait(idxb, rs, rd, sem):
        c1, c2 = gather_copies(idxb, rs, rd, sem)
        c1.wait()
        c2.wait()

    def compute(i, rs, rd, ov):
        b = wid + i * NW
        et = b // NBLK_ET
        wbase = et * IN_DIM
        zero = jnp.zeros((16,), jnp.float32)

        @plsc.parallel_loop(0, IN_DIM // 16, carry=(zero,) * NG)
        def dloop(dg, accs):
            accs = list(accs)
            wv = w_v[pl.ds(wbase + dg * 16, 16)]
            cb0 = jnp.zeros((16,), jnp.int32) + dg * 16
            for j in range(16):
                colv = cb0 + j
                wb = wv[j]
                for g in range(NG):
                    sv = plsc.load_gather(rs, [rids[g], colv])
                    tv = plsc.load_gather(rd, [rids[g], colv])
                    accs[g] = accs[g] + sv * tv * wb
            return tuple(accs)

        accs = dloop
        for g in range(NG):
            acc = accs[g]
            sgm = 1.0 / (1.0 + jnp.exp(-acc))
            ov[pl.ds(g * 16, 16)] = acc + sig * (sgm - acc)

    def step(i, idxP, rsP, rdP, ovP, siP, srP, soP,
             idxO, rsO, rdO, siO, srO):
        gather_wait(idxP, rsP, rdP, srP)

        @pl.when(i + 2 < n)
        def _():
            idx_copy(i + 2, idxP, siP).start()

        @pl.when(i + 1 < n)
        def _():
            idx_copy(i + 1, idxO, siO).wait()
            gather_start(idxO, rsO, rdO, srO)

        @pl.when(i >= 2)
        def _():
            out_copy(i, ovP, soP).wait()

        compute(i, rsP, rdP, ovP)
        out_copy(i, ovP, soP).start()

    pltpu.sync_copy(idx_hbm.at[pl.ds(wid * 2, 2)], idx0)
    gather_start(idx0, rs0, rd0, sr0)
    idx_copy(1, idx1, si1).start()

    def pair(p, carry):
        i0 = p * 2
        step(i0, idx0, rs0, rd0, ov0, si0, sr0, so0,
             idx1, rs1, rd1, si1, sr1)

        @pl.when(i0 + 1 < n)
        def _():
            step(i0 + 1, idx1, rs1, rd1, ov1, si1, sr1, so1,
                 idx0, rs0, rd0, si0, sr0)
        return carry

    lax.fori_loop(0, (n + 1) // 2, pair, 0)

    out_copy(0, ov0, so0).wait()
    out_copy(0, ov1, so1).wait()


@jax.jit
def _decode_all(z, edge_index, weight, sig_f32):
    mesh = plsc.VectorSubcoreMesh(core_axis_name="c", subcore_axis_name="s")
    run = functools.partial(
        pl.kernel,
        mesh=mesh,
        out_type=jax.ShapeDtypeStruct((NUM_ET * N_EDGES,), jnp.float32),
        scratch_types=[
            pltpu.VMEM((2, CB), jnp.int32),
            pltpu.VMEM((2, CB), jnp.int32),
            pltpu.VMEM((CB, IN_DIM), jnp.float32),
            pltpu.VMEM((CB, IN_DIM), jnp.float32),
            pltpu.VMEM((CB, IN_DIM), jnp.float32),
            pltpu.VMEM((CB, IN_DIM), jnp.float32),
            pltpu.VMEM((CB,), jnp.float32),
            pltpu.VMEM((CB,), jnp.float32),
            pltpu.VMEM((NUM_ET * IN_DIM,), jnp.float32),
            pltpu.VMEM((16,), jnp.float32),
            pltpu.SemaphoreType.DMA,
            pltpu.SemaphoreType.DMA,
            pltpu.SemaphoreType.DMA,
            pltpu.SemaphoreType.DMA,
            pltpu.SemaphoreType.DMA,
            pltpu.SemaphoreType.DMA,
        ],
        compiler_params=pltpu.CompilerParams(needs_layout_passes=False),
    )(_sc_kernel_body)
    idx = edge_index.reshape(NUM_ET, 2, NBLK_ET, CB).transpose(0, 2, 1, 3)
    idx = idx + (jnp.arange(NUM_ET, dtype=jnp.int32) * N_NODES
                 ).reshape(NUM_ET, 1, 1, 1)
    idx = idx.reshape(NB * 2, CB)
    sig_vec = jnp.full((16,), 1.0, jnp.float32) * sig_f32
    return run(z.reshape(NUM_ET * N_NODES, IN_DIM), idx,
               weight.reshape(-1), sig_vec)


def kernel(z, edge_index, weight, sigmoid):
    sig_f32 = jnp.asarray(sigmoid, jnp.float32)
    out = _decode_all(z, edge_index, weight, sig_f32)
    per_et = tuple(out[et * N_EDGES:(et + 1) * N_EDGES] for et in range(NUM_ET))
    return (per_et, out)


# speedup vs baseline: 3.3189x; 3.3189x over previous
import functools

import jax
import jax.numpy as jnp
from jax import lax
from jax.experimental import pallas as pl
from jax.experimental.pallas import tpu as pltpu
from jax.experimental.pallas import tpu_sc as plsc

NUM_ET = 4
N_NODES = 100000
N_EDGES = 150000
IN_DIM = 128

NW = 32
CB = 80
NBLK_ET = N_EDGES // CB
NB = NUM_ET * NBLK_ET
NG = CB // 16


def _sc_kernel_body(z_all, idx_hbm, w_hbm, sig_hbm, out_hbm,
                    idx0, idx1, rs0, rd0, rs1, rd1, ov0, ov1, w_v, sig_v,
                    si0, si1, sr0, sr1, so0, so1):
    cid = lax.axis_index("c")
    sid = lax.axis_index("s")
    wid = sid * 2 + cid

    pltpu.sync_copy(w_hbm, w_v)
    pltpu.sync_copy(sig_hbm, sig_v)
    sig = sig_v[...]
    lane = lax.iota(jnp.int32, 16)
    rids = [lane + g * 16 for g in range(NG)]

    n = (NB - wid + NW - 1) // NW

    def idx_copy(i, idxb, sem):
        b = wid + i * NW
        return pltpu.make_async_copy(
            idx_hbm.at[pl.ds(b * 2, 2)], idxb, sem)

    def gather_copies(idxb, rs, rd, sem):
        return (pltpu.make_async_copy(z_all.at[idxb.at[0]], rs, sem),
                pltpu.make_async_copy(z_all.at[idxb.at[1]], rd, sem))

    def out_copy(i, ov, sem):
        b = wid + i * NW
        return pltpu.make_async_copy(ov, out_hbm.at[pl.ds(b * CB, CB)], sem)

    def gather_start(idxb, rs, rd, sem):
        c1, c2 = gather_copies(idxb, rs, rd, sem)
        c1.start()
        c2.start()

    def gather_wait(idxb, rs, rd, sem):
        c1, c2 = gather_copies(idxb, rs, rd, sem)
        c1.wait()
        c2.wait()

    def compute(i, rs, rd, ov):
        b = wid + i * NW
        et = b // NBLK_ET
        wbase = et * IN_DIM
        zero = jnp.zeros((16,), jnp.float32)

        @plsc.parallel_loop(0, IN_DIM // 16, carry=(zero,) * NG)
        def dloop(dg, accs):
            accs = list(accs)
            cb0 = jnp.zeros((16,), jnp.int32) + dg * 16
            wb0 = cb0 + wbase
            for j in range(16):
                rot = (lane + j) & 15
                colv = cb0 + rot
                wv = plsc.load_gather(w_v, [wb0 + rot])
                for g in range(NG):
                    sv = plsc.load_gather(rs, [rids[g], colv])
                    tv = plsc.load_gather(rd, [rids[g], colv])
                    accs[g] = accs[g] + sv * tv * wv
            return tuple(accs)

        accs = dloop
        for g in range(NG):
            acc = accs[g]
            sgm = 1.0 / (1.0 + jnp.exp(-acc))
            ov[pl.ds(g * 16, 16)] = acc + sig * (sgm - acc)

    def step(i, idxP, rsP, rdP, ovP, siP, srP, soP,
             idxO, rsO, rdO, siO, srO):
        gather_wait(idxP, rsP, rdP, srP)

        @pl.when(i + 2 < n)
        def _():
            idx_copy(i + 2, idxP, siP).start()

        @pl.when(i + 1 < n)
        def _():
            idx_copy(i + 1, idxO, siO).wait()
            gather_start(idxO, rsO, rdO, srO)

        @pl.when(i >= 2)
        def _():
            out_copy(i, ovP, soP).wait()

        compute(i, rsP, rdP, ovP)
        out_copy(i, ovP, soP).start()

    pltpu.sync_copy(idx_hbm.at[pl.ds(wid * 2, 2)], idx0)
    gather_start(idx0, rs0, rd0, sr0)
    idx_copy(1, idx1, si1).start()

    def pair(p, carry):
        i0 = p * 2
        step(i0, idx0, rs0, rd0, ov0, si0, sr0, so0,
             idx1, rs1, rd1, si1, sr1)

        @pl.when(i0 + 1 < n)
        def _():
            step(i0 + 1, idx1, rs1, rd1, ov1, si1, sr1, so1,
                 idx0, rs0, rd0, si0, sr0)
        return carry

    lax.fori_loop(0, (n + 1) // 2, pair, 0)

    out_copy(0, ov0, so0).wait()
    out_copy(0, ov1, so1).wait()


@jax.jit
def _decode_all(z, edge_index, weight, sig_f32):
    mesh = plsc.VectorSubcoreMesh(core_axis_name="c", subcore_axis_name="s")
    run = functools.partial(
        pl.kernel,
        mesh=mesh,
        out_type=jax.ShapeDtypeStruct((NUM_ET * N_EDGES,), jnp.float32),
        scratch_types=[
            pltpu.VMEM((2, CB), jnp.int32),
            pltpu.VMEM((2, CB), jnp.int32),
            pltpu.VMEM((CB, IN_DIM), jnp.float32),
            pltpu.VMEM((CB, IN_DIM), jnp.float32),
            pltpu.VMEM((CB, IN_DIM), jnp.float32),
            pltpu.VMEM((CB, IN_DIM), jnp.float32),
            pltpu.VMEM((CB,), jnp.float32),
            pltpu.VMEM((CB,), jnp.float32),
            pltpu.VMEM((NUM_ET * IN_DIM,), jnp.float32),
            pltpu.VMEM((16,), jnp.float32),
            pltpu.SemaphoreType.DMA,
            pltpu.SemaphoreType.DMA,
            pltpu.SemaphoreType.DMA,
            pltpu.SemaphoreType.DMA,
            pltpu.SemaphoreType.DMA,
            pltpu.SemaphoreType.DMA,
        ],
        compiler_params=pltpu.CompilerParams(needs_layout_passes=False),
    )(_sc_kernel_body)
    idx = edge_index.reshape(NUM_ET, 2, NBLK_ET, CB).transpose(0, 2, 1, 3)
    idx = idx + (jnp.arange(NUM_ET, dtype=jnp.int32) * N_NODES
                 ).reshape(NUM_ET, 1, 1, 1)
    idx = idx.reshape(NB * 2, CB)
    sig_vec = jnp.full((16,), 1.0, jnp.float32) * sig_f32
    return run(z.reshape(NUM_ET * N_NODES, IN_DIM), idx,
               weight.reshape(-1), sig_vec)


def kernel(z, edge_index, weight, sigmoid):
    sig_f32 = jnp.asarray(sigmoid, jnp.float32)
    out = _decode_all(z, edge_index, weight, sig_f32)
    per_et = tuple(out[et * N_EDGES:(et + 1) * N_EDGES] for et in range(NUM_ET))
    return (per_et, out)
